# Initial kernel scaffold; baseline (speedup 1.0000x reference)
import jax
import jax.numpy as jnp
from jax.experimental import pallas as pl


def kernel(learned_means, true_means, X_train):
    raise NotImplementedError("write your pallas kernel here")


# speedup vs baseline: 5.3675x; 5.3675x over previous
import functools

import jax
import jax.numpy as jnp
from jax.experimental import pallas as pl
from jax.experimental.pallas import tpu as pltpu

_T = 1.0 / 3.0
_FBIG = 3e38
_PCT_LOC = []
for _q in (10.0, 25.0, 50.0, 75.0, 90.0):
    _loc = _q / 100.0 * 1023.0
    _lo = int(_loc)
    _PCT_LOC.append((_lo, _loc - _lo))


def _top2_block_kernel(x_ref, lm_ref, out_ref, *, n_valid, block_rows):
    i = pl.program_id(0)
    xb = x_ref[...]
    lm = lm_ref[...] * jnp.float32(-2.0)
    g = jax.lax.dot_general(
        xb, lm, (((1,), (1,)), ((), ())), preferred_element_type=jnp.float32
    )
    y2 = jnp.sum(xb * xb, axis=1, keepdims=True)
    d = y2 + g
    if n_valid % block_rows != 0:
        row = i * block_rows + jax.lax.broadcasted_iota(
            jnp.int32, (block_rows, 1), 0
        )
        d = jnp.where(row < n_valid, d, _FBIG)
    m1 = jnp.min(d, axis=0, keepdims=True)
    eq = d == m1
    cnt = jnp.sum(eq.astype(jnp.float32), axis=0, keepdims=True)
    m2m = jnp.min(jnp.where(eq, _FBIG, d), axis=0, keepdims=True)
    m2 = jnp.where(cnt > 1.0, m1, m2m)

    @pl.when(i == 0)
    def _init():
        out_ref[0:1, :] = m1
        out_ref[1:2, :] = m2

    @pl.when(i > 0)
    def _merge():
        a1 = out_ref[0:1, :]
        a2 = out_ref[1:2, :]
        out_ref[0:1, :] = jnp.minimum(a1, m1)
        out_ref[1:2, :] = jnp.minimum(jnp.maximum(a1, m1), jnp.minimum(a2, m2))


def _top2_sqdist(dataset, lm, block_rows):
    n = dataset.shape[0]
    n_pad = -(-n // block_rows) * block_rows
    if n_pad != n:
        dataset = jnp.pad(dataset, ((0, n_pad - n), (0, 0)))
    grid = n_pad // block_rows
    return pl.pallas_call(
        functools.partial(
            _top2_block_kernel, n_valid=n, block_rows=block_rows
        ),
        grid=(grid,),
        in_specs=[
            pl.BlockSpec((block_rows, 16), lambda i: (i, 0)),
            pl.BlockSpec((1024, 16), lambda i: (0, 0)),
        ],
        out_specs=pl.BlockSpec((2, 1024), lambda i: (0, 0)),
        out_shape=jax.ShapeDtypeStruct((2, 1024), jnp.float32),
        compiler_params=pltpu.CompilerParams(
            dimension_semantics=("arbitrary",),
        ),
    )(dataset, lm)


def _rank_order_stats(v_row, v_col, lt_ij):
    lt = (v_row < v_col).astype(jnp.float32)
    eq = ((v_row == v_col) & lt_ij).astype(jnp.float32)
    rank = jnp.sum(lt, axis=1, keepdims=True) + jnp.sum(eq, axis=1, keepdims=True)
    out = []
    for lo, frac in _PCT_LOC:
        v_lo = jnp.sum(jnp.where(rank == lo, v_col, 0.0))
        v_hi = jnp.sum(jnp.where(rank == lo + 1, v_col, 0.0))
        out.append(v_lo * jnp.float32(1.0 - frac) + v_hi * jnp.float32(frac))
    return out


def _stats_kernel(mm_ref, ss_ref, mmt_ref, sst_ref, x2r_ref, x2c_ref, out_ref):
    x2 = x2r_ref[...]
    x2c = x2c_ref[...]
    eps = jnp.float32(1e-12)
    dm1 = jnp.sqrt(jnp.maximum(mm_ref[0:1, :] + x2, eps))
    dm2 = jnp.sqrt(jnp.maximum(mm_ref[1:2, :] + x2, eps))
    ds1 = jnp.sqrt(jnp.maximum(ss_ref[0:1, :] + x2, eps))
    ds2 = jnp.sqrt(jnp.maximum(ss_ref[1:2, :] + x2, eps))
    dm1c = jnp.sqrt(jnp.maximum(mmt_ref[:, 0:1] + x2c, eps))
    ds1c = jnp.sqrt(jnp.maximum(sst_ref[:, 0:1] + x2c, eps))

    t = jnp.float32(_T)
    near_true = (dm1 < t * ds1) & (dm1 < t * dm2)
    near_samp = (ds1 < t * dm1) & (ds1 < t * ds2)

    vals = [
        jnp.sum(near_true.astype(jnp.float32)),
        jnp.sum(near_samp.astype(jnp.float32)),
        jnp.mean(dm1),
        jnp.mean(ds1),
        jnp.mean(dm2),
        jnp.mean(ds2),
    ]

    ii = jax.lax.broadcasted_iota(jnp.int32, (1024, 1024), 0)
    jj = jax.lax.broadcasted_iota(jnp.int32, (1024, 1024), 1)
    lt_ij = jj < ii
    vals += _rank_order_stats(dm1, dm1c, lt_ij)
    vals += _rank_order_stats(ds1, ds1c, lt_ij)
    lane = jax.lax.broadcasted_iota(jnp.int32, (1, 128), 1)
    row = jnp.zeros((1, 128), jnp.float32)
    for k, v in enumerate(vals):
        row = jnp.where(lane == k, v, row)
    out_ref[...] = row


def _stats(mm, ss, x2):
    return pl.pallas_call(
        _stats_kernel,
        in_specs=[
            pl.BlockSpec((2, 1024), lambda: (0, 0)),
            pl.BlockSpec((2, 1024), lambda: (0, 0)),
            pl.BlockSpec((1024, 2), lambda: (0, 0)),
            pl.BlockSpec((1024, 2), lambda: (0, 0)),
            pl.BlockSpec((1, 1024), lambda: (0, 0)),
            pl.BlockSpec((1024, 1), lambda: (0, 0)),
        ],
        out_specs=pl.BlockSpec((1, 128), lambda: (0, 0)),
        out_shape=jax.ShapeDtypeStruct((1, 128), jnp.float32),
    )(mm, ss, mm.T, ss.T, x2[None, :], x2[:, None])


def kernel(learned_means, true_means, X_train):
    mm = _top2_sqdist(true_means, learned_means, block_rows=1024)
    ss = _top2_sqdist(X_train, learned_means, block_rows=1000)
    x2 = jnp.sum(learned_means * learned_means, axis=1)
    s = _stats(mm, ss, x2)[0]
    return (
        s[0].astype(jnp.int32),
        s[1].astype(jnp.int32),
        s[2],
        s[3],
        s[4],
        s[5],
        s[6:11],
        s[11:16],
    )
